# baseline (speedup 1.0000x reference)
import functools

import jax
import jax.numpy as jnp
from jax.experimental import pallas as pl
from jax.experimental.pallas import tpu as pltpu
from jax.experimental.pallas import tpu_sc as plsc

_E = 64


def kernel(loads, capacities):
    mesh = plsc.ScalarSubcoreMesh(axis_name="c", num_cores=1)

    @functools.partial(
        pl.kernel,
        out_type=jax.ShapeDtypeStruct((2, _E), jnp.float32),
        mesh=mesh,
        scratch_types=[pltpu.SemaphoreType.DMA, pltpu.SemaphoreType.DMA],
    )
    def _router_stats(loads_hbm, caps_hbm, out_hbm, sem0, sem1):
        c0 = pltpu.async_copy(loads_hbm, out_hbm.at[0], sem0)
        c1 = pltpu.async_copy(caps_hbm, out_hbm.at[1], sem1)
        c0.wait()
        c1.wait()

    return _router_stats(loads, capacities)
